# baseline (speedup 1.0000x reference)
import jax
import jax.numpy as jnp
from jax.experimental import pallas as pl
from jax.experimental.pallas import tpu as pltpu
from jax.experimental.pallas import tpu_sc as plsc

IN_DIM = 128
OUT_DIM = 256
MEM_LEN = 100000
BETA = 2000.0

BLK = 5000
NBLK = MEM_LEN // BLK

SC_CORES = 2
SC_SUBCORES = 16
SC_WORKERS = SC_CORES * SC_SUBCORES
SC_BASE_ROWS = 3120
SC_SIZES = [400] * 7 + [320]
SC_OFFS = [sum(SC_SIZES[:k]) for k in range(len(SC_SIZES))]
SC_TAIL_START = SC_BASE_ROWS * SC_WORKERS
SC_TAIL_WORKERS = (MEM_LEN - SC_TAIL_START) // 8


def _tc_body(pos_ref, x_ref, mean_ref, std_ref, w_ref, b_ref, mem_ref,
             loss_ref, out_mem_ref, enc_ref, min_ref):
    i = pl.program_id(0)

    @pl.when(i == 0)
    def _encode():
        xv = x_ref[...]
        std = std_ref[...]
        new = jnp.where(std == 0.0, 0.0, (xv - mean_ref[...]) / std)
        logits = jnp.dot(new, w_ref[...],
                         preferred_element_type=jnp.float32) + b_ref[...]
        m = jnp.max(logits)
        lse = jnp.log(jnp.sum(jnp.exp(logits - m))) + m
        enc_ref[...] = logits - lse
        min_ref[0] = jnp.inf

    @pl.when(i < NBLK)
    def _stream():
        blk = mem_ref[...]
        out_mem_ref[...] = blk
        d = jnp.sum(jnp.abs(blk - enc_ref[...]), axis=1)
        min_ref[0] = jnp.minimum(min_ref[0], jnp.min(d))

    @pl.when(i == NBLK)
    def _fixup():
        loss = min_ref[0]
        loss_ref[...] = jnp.full((1, 1), loss, jnp.float32)
        do_update = loss <= BETA
        r = pos_ref[0] % BLK
        row_sel = jax.lax.broadcasted_iota(jnp.int32, (BLK, 1), 0) == r
        sel = jnp.logical_and(do_update, row_sel)
        out_mem_ref[...] = jnp.where(sel, enc_ref[...], mem_ref[...])


def _tc_pass(pos, x, mean2, std2, W_enc, b2, memory):
    def big_map(i, p):
        return (jnp.where(i < NBLK, i, p[0] // BLK), 0)

    def const_map(i, p):
        return (0, 0)

    grid_spec = pltpu.PrefetchScalarGridSpec(
        num_scalar_prefetch=1,
        grid=(NBLK + 1,),
        in_specs=[
            pl.BlockSpec((1, IN_DIM), const_map),
            pl.BlockSpec((1, IN_DIM), const_map),
            pl.BlockSpec((1, IN_DIM), const_map),
            pl.BlockSpec((IN_DIM, OUT_DIM), const_map),
            pl.BlockSpec((1, OUT_DIM), const_map),
            pl.BlockSpec((BLK, OUT_DIM), big_map),
        ],
        out_specs=[
            pl.BlockSpec((1, 1), const_map),
            pl.BlockSpec((BLK, OUT_DIM), big_map),
        ],
        scratch_shapes=[
            pltpu.VMEM((1, OUT_DIM), jnp.float32),
            pltpu.SMEM((1,), jnp.float32),
        ],
    )

    return pl.pallas_call(
        _tc_body,
        grid_spec=grid_spec,
        out_shape=[
            jax.ShapeDtypeStruct((1, 1), jnp.float32),
            jax.ShapeDtypeStruct((MEM_LEN, OUT_DIM), jnp.float32),
        ],
        compiler_params=pltpu.CompilerParams(
            dimension_semantics=("arbitrary",),
        ),
    )(pos, x, mean2, std2, W_enc, b2, memory)


def _sc_copy(mem_data):
    mesh = plsc.VectorSubcoreMesh(core_axis_name="c", subcore_axis_name="s")

    @pl.kernel(
        out_type=jax.ShapeDtypeStruct((MEM_LEN, IN_DIM), jnp.float32),
        mesh=mesh,
        scratch_types=[
            pltpu.VMEM((SC_SIZES[0], IN_DIM), jnp.float32),
            pltpu.VMEM((SC_SIZES[0], IN_DIM), jnp.float32),
            pltpu.SemaphoreType.DMA,
            pltpu.SemaphoreType.DMA,
            pltpu.SemaphoreType.DMA,
            pltpu.SemaphoreType.DMA,
        ],
    )
    def sc_kernel(md_hbm, o_hbm, buf0, buf1, sin0, sin1, sout0, sout1):
        core = jax.lax.axis_index("c")
        sub = jax.lax.axis_index("s")
        w = core * SC_SUBCORES + sub
        base = w * SC_BASE_ROWS
        bufs, sins, souts = [buf0, buf1], [sin0, sin1], [sout0, sout1]
        nch = len(SC_SIZES)

        def bslice(b, sz):
            return bufs[b] if sz == SC_SIZES[0] else bufs[b].at[pl.ds(0, sz), :]

        in_c = [None, None]
        out_c = [None, None]
        in_c[0] = pltpu.async_copy(
            md_hbm.at[pl.ds(base + SC_OFFS[0], SC_SIZES[0]), :],
            bslice(0, SC_SIZES[0]), sin0)
        for k in range(nch):
            b, nb = k & 1, 1 - (k & 1)
            if k + 1 < nch:
                if out_c[nb] is not None:
                    out_c[nb].wait()
                in_c[nb] = pltpu.async_copy(
                    md_hbm.at[pl.ds(base + SC_OFFS[k + 1], SC_SIZES[k + 1]), :],
                    bslice(nb, SC_SIZES[k + 1]), sins[nb])
            in_c[b].wait()
            out_c[b] = pltpu.async_copy(
                bslice(b, SC_SIZES[k]),
                o_hbm.at[pl.ds(base + SC_OFFS[k], SC_SIZES[k]), :], souts[b])
        out_c[0].wait()
        out_c[1].wait()

        @pl.when(w < SC_TAIL_WORKERS)
        def _tail():
            s = SC_TAIL_START + w * 8
            pltpu.sync_copy(md_hbm.at[pl.ds(s, 8), :], buf0.at[pl.ds(0, 8), :])
            pltpu.sync_copy(buf0.at[pl.ds(0, 8), :], o_hbm.at[pl.ds(s, 8), :])

    return sc_kernel(mem_data)


def _md_fixup_body(md_any, loss_ref, x_ref, pos_ref, out_any, row_vmem, sem):
    @pl.when(loss_ref[0] <= BETA)
    def _():
        row_vmem[...] = x_ref[...]
        pltpu.async_copy(
            row_vmem, out_any.at[pl.ds(pos_ref[0], 1), :], sem,
        ).wait()


def _md_fixup(md_copied, loss2d, x, pos):
    return pl.pallas_call(
        _md_fixup_body,
        grid=(),
        in_specs=[
            pl.BlockSpec(memory_space=pltpu.MemorySpace.HBM),
            pl.BlockSpec(memory_space=pltpu.SMEM),
            pl.BlockSpec(memory_space=pltpu.VMEM),
            pl.BlockSpec(memory_space=pltpu.SMEM),
        ],
        out_specs=pl.BlockSpec(memory_space=pltpu.MemorySpace.HBM),
        out_shape=jax.ShapeDtypeStruct((MEM_LEN, IN_DIM), jnp.float32),
        scratch_shapes=[
            pltpu.VMEM((1, IN_DIM), jnp.float32),
            pltpu.SemaphoreType.DMA,
        ],
        input_output_aliases={0: 0},
    )(md_copied, loss2d.reshape(1), x, pos)


def kernel(x, mean, std, W_enc, b_enc, memory, mem_data, count):
    pos = jnp.asarray(count % MEM_LEN, jnp.int32).reshape(1)
    mean2 = mean.reshape(1, IN_DIM)
    std2 = std.reshape(1, IN_DIM)
    b2 = b_enc.reshape(1, OUT_DIM)

    md_copied = _sc_copy(mem_data)
    loss2d, new_memory = _tc_pass(pos, x, mean2, std2, W_enc, b2, memory)
    new_mem_data = _md_fixup(md_copied, loss2d, x, pos)

    return loss2d.reshape(()), new_memory, new_mem_data


# speedup vs baseline: 1.1922x; 1.1922x over previous
import jax
import jax.numpy as jnp
from jax.experimental import pallas as pl
from jax.experimental.pallas import tpu as pltpu

IN_DIM = 128
OUT_DIM = 256
MEM_LEN = 100000
BETA = 2000.0

BLK = 5000
NBLK = MEM_LEN // BLK


def _body(x_ref, mean_ref, std_ref, w_ref, b_ref, mem_ref, md_ref,
          loss_ref, enc_out_ref, out_mem_ref, out_md_ref, enc_ref, min_ref):
    i = pl.program_id(0)

    @pl.when(i == 0)
    def _encode():
        xv = x_ref[...]
        std = std_ref[...]
        new = jnp.where(std == 0.0, 0.0, (xv - mean_ref[...]) / std)
        logits = jnp.dot(new, w_ref[...],
                         preferred_element_type=jnp.float32) + b_ref[...]
        m = jnp.max(logits)
        lse = jnp.log(jnp.sum(jnp.exp(logits - m))) + m
        enc = logits - lse
        enc_ref[...] = enc
        enc_out_ref[...] = enc
        min_ref[0] = jnp.inf

    blk = mem_ref[...]
    out_mem_ref[...] = blk
    out_md_ref[...] = md_ref[...]
    d = jnp.sum(jnp.abs(blk - enc_ref[...]), axis=1)
    min_ref[0] = jnp.minimum(min_ref[0], jnp.min(d))

    @pl.when(i == NBLK - 1)
    def _emit_loss():
        loss_ref[...] = jnp.full((1, 1), min_ref[0], jnp.float32)


def _row_fixup_body(dst_any, loss_ref, row_ref, pos_ref, out_any,
                    row_vmem, sem):
    @pl.when(loss_ref[0] <= BETA)
    def _():
        row_vmem[...] = row_ref[...]
        pltpu.async_copy(
            row_vmem, out_any.at[pl.ds(pos_ref[0], 1), :], sem,
        ).wait()


def _row_fixup(dst, loss1, row, pos, dim):
    return pl.pallas_call(
        _row_fixup_body,
        grid=(),
        in_specs=[
            pl.BlockSpec(memory_space=pltpu.MemorySpace.HBM),
            pl.BlockSpec(memory_space=pltpu.SMEM),
            pl.BlockSpec(memory_space=pltpu.VMEM),
            pl.BlockSpec(memory_space=pltpu.SMEM),
        ],
        out_specs=pl.BlockSpec(memory_space=pltpu.MemorySpace.HBM),
        out_shape=jax.ShapeDtypeStruct((MEM_LEN, dim), jnp.float32),
        scratch_shapes=[
            pltpu.VMEM((1, dim), jnp.float32),
            pltpu.SemaphoreType.DMA,
        ],
        input_output_aliases={0: 0},
    )(dst, loss1, row, pos)


def kernel(x, mean, std, W_enc, b_enc, memory, mem_data, count):
    pos = jnp.asarray(count % MEM_LEN, jnp.int32).reshape(1)
    mean2 = mean.reshape(1, IN_DIM)
    std2 = std.reshape(1, IN_DIM)
    b2 = b_enc.reshape(1, OUT_DIM)

    def big_map(i):
        return (i, 0)

    def const_map(i):
        return (0, 0)

    loss2d, enc, mem_copied, md_copied = pl.pallas_call(
        _body,
        grid=(NBLK,),
        in_specs=[
            pl.BlockSpec((1, IN_DIM), const_map),
            pl.BlockSpec((1, IN_DIM), const_map),
            pl.BlockSpec((1, IN_DIM), const_map),
            pl.BlockSpec((IN_DIM, OUT_DIM), const_map),
            pl.BlockSpec((1, OUT_DIM), const_map),
            pl.BlockSpec((BLK, OUT_DIM), big_map),
            pl.BlockSpec((BLK, IN_DIM), big_map),
        ],
        out_specs=[
            pl.BlockSpec((1, 1), const_map),
            pl.BlockSpec((1, OUT_DIM), const_map),
            pl.BlockSpec((BLK, OUT_DIM), big_map),
            pl.BlockSpec((BLK, IN_DIM), big_map),
        ],
        out_shape=[
            jax.ShapeDtypeStruct((1, 1), jnp.float32),
            jax.ShapeDtypeStruct((1, OUT_DIM), jnp.float32),
            jax.ShapeDtypeStruct((MEM_LEN, OUT_DIM), jnp.float32),
            jax.ShapeDtypeStruct((MEM_LEN, IN_DIM), jnp.float32),
        ],
        scratch_shapes=[
            pltpu.VMEM((1, OUT_DIM), jnp.float32),
            pltpu.SMEM((1,), jnp.float32),
        ],
        compiler_params=pltpu.CompilerParams(
            dimension_semantics=("arbitrary",),
        ),
    )(x, mean2, std2, W_enc, b2, memory, mem_data)

    loss1 = loss2d.reshape(1)
    new_memory = _row_fixup(mem_copied, loss1, enc, pos, OUT_DIM)
    new_mem_data = _row_fixup(md_copied, loss1, x, pos, IN_DIM)

    return loss2d.reshape(()), new_memory, new_mem_data


# speedup vs baseline: 1.2116x; 1.0163x over previous
import jax
import jax.numpy as jnp
from jax.experimental import pallas as pl
from jax.experimental.pallas import tpu as pltpu

IN_DIM = 128
OUT_DIM = 256
MEM_LEN = 100000
BETA = 2000.0

BLK = 5000
NBLK = MEM_LEN // BLK


def _body(x_ref, mean_ref, std_ref, w_ref, b_ref, mem_ref, md_ref,
          loss_ref, enc_out_ref, out_mem_ref, out_md_ref, enc_ref, min_ref):
    i = pl.program_id(0)

    @pl.when(i == 0)
    def _encode():
        xv = x_ref[...]
        std = std_ref[...]
        new = jnp.where(std == 0.0, 0.0, (xv - mean_ref[...]) / std)
        logits = jnp.dot(new, w_ref[...],
                         preferred_element_type=jnp.float32) + b_ref[...]
        m = jnp.max(logits)
        lse = jnp.log(jnp.sum(jnp.exp(logits - m))) + m
        enc = logits - lse
        enc_ref[...] = enc
        enc_out_ref[...] = enc
        min_ref[0] = jnp.inf

    blk = mem_ref[...]
    out_mem_ref[...] = blk
    out_md_ref[...] = md_ref[...]
    d = jnp.sum(jnp.abs(blk - enc_ref[...]), axis=1)
    min_ref[0] = jnp.minimum(min_ref[0], jnp.min(d))

    @pl.when(i == NBLK - 1)
    def _emit_loss():
        loss_ref[...] = jnp.full((1, 1), min_ref[0], jnp.float32)


def _fixup_body(mem_any, md_any, loss_ref, enc_ref, x_ref, pos_ref,
                out_mem_any, out_md_any, enc_vmem, x_vmem, sem_a, sem_b):
    @pl.when(loss_ref[0] <= BETA)
    def _():
        enc_vmem[...] = enc_ref[...]
        x_vmem[...] = x_ref[...]
        c1 = pltpu.async_copy(
            enc_vmem, out_mem_any.at[pl.ds(pos_ref[0], 1), :], sem_a)
        c2 = pltpu.async_copy(
            x_vmem, out_md_any.at[pl.ds(pos_ref[0], 1), :], sem_b)
        c1.wait()
        c2.wait()


def _fixup(mem_copied, md_copied, loss1, enc, x, pos):
    return pl.pallas_call(
        _fixup_body,
        grid=(),
        in_specs=[
            pl.BlockSpec(memory_space=pltpu.MemorySpace.HBM),
            pl.BlockSpec(memory_space=pltpu.MemorySpace.HBM),
            pl.BlockSpec(memory_space=pltpu.SMEM),
            pl.BlockSpec(memory_space=pltpu.VMEM),
            pl.BlockSpec(memory_space=pltpu.VMEM),
            pl.BlockSpec(memory_space=pltpu.SMEM),
        ],
        out_specs=[
            pl.BlockSpec(memory_space=pltpu.MemorySpace.HBM),
            pl.BlockSpec(memory_space=pltpu.MemorySpace.HBM),
        ],
        out_shape=[
            jax.ShapeDtypeStruct((MEM_LEN, OUT_DIM), jnp.float32),
            jax.ShapeDtypeStruct((MEM_LEN, IN_DIM), jnp.float32),
        ],
        scratch_shapes=[
            pltpu.VMEM((1, OUT_DIM), jnp.float32),
            pltpu.VMEM((1, IN_DIM), jnp.float32),
            pltpu.SemaphoreType.DMA,
            pltpu.SemaphoreType.DMA,
        ],
        input_output_aliases={0: 0, 1: 1},
    )(mem_copied, md_copied, loss1, enc, x, pos)


def kernel(x, mean, std, W_enc, b_enc, memory, mem_data, count):
    pos = jnp.asarray(count % MEM_LEN, jnp.int32).reshape(1)
    mean2 = mean.reshape(1, IN_DIM)
    std2 = std.reshape(1, IN_DIM)
    b2 = b_enc.reshape(1, OUT_DIM)

    def big_map(i):
        return (i, 0)

    def const_map(i):
        return (0, 0)

    loss2d, enc, mem_copied, md_copied = pl.pallas_call(
        _body,
        grid=(NBLK,),
        in_specs=[
            pl.BlockSpec((1, IN_DIM), const_map),
            pl.BlockSpec((1, IN_DIM), const_map),
            pl.BlockSpec((1, IN_DIM), const_map),
            pl.BlockSpec((IN_DIM, OUT_DIM), const_map),
            pl.BlockSpec((1, OUT_DIM), const_map),
            pl.BlockSpec((BLK, OUT_DIM), big_map),
            pl.BlockSpec((BLK, IN_DIM), big_map),
        ],
        out_specs=[
            pl.BlockSpec((1, 1), const_map),
            pl.BlockSpec((1, OUT_DIM), const_map),
            pl.BlockSpec((BLK, OUT_DIM), big_map),
            pl.BlockSpec((BLK, IN_DIM), big_map),
        ],
        out_shape=[
            jax.ShapeDtypeStruct((1, 1), jnp.float32),
            jax.ShapeDtypeStruct((1, OUT_DIM), jnp.float32),
            jax.ShapeDtypeStruct((MEM_LEN, OUT_DIM), jnp.float32),
            jax.ShapeDtypeStruct((MEM_LEN, IN_DIM), jnp.float32),
        ],
        scratch_shapes=[
            pltpu.VMEM((1, OUT_DIM), jnp.float32),
            pltpu.SMEM((1,), jnp.float32),
        ],
        compiler_params=pltpu.CompilerParams(
            dimension_semantics=("arbitrary",),
        ),
    )(x, mean2, std2, W_enc, b2, memory, mem_data)

    loss1 = loss2d.reshape(1)
    new_memory, new_mem_data = _fixup(mem_copied, md_copied, loss1, enc, x, pos)

    return loss2d.reshape(()), new_memory, new_mem_data
